# baseline (speedup 1.0000x reference)
import jax
import jax.numpy as jnp
from jax.experimental import pallas as pl
from jax.experimental.pallas import tpu as pltpu

_B = 128
_N = 32768
_CW = 8192
_NC = _N // _CW
_NB = 4
_DEPTH = 3


def _body(x_hbm, m8_hbm, o_hbm, xb, mbuf, ob, sx, sm, so):

    def in_copy(c):
        slot = c % _NB
        return pltpu.make_async_copy(
            x_hbm.at[:, pl.ds(c * _CW, _CW)], xb.at[slot], sx.at[slot])

    def out_copy(c):
        slot = c % _NB
        return pltpu.make_async_copy(
            ob.at[slot], o_hbm.at[:, pl.ds(c * _CW, _CW)], so.at[slot])

    mask_copy = pltpu.make_async_copy(m8_hbm, mbuf, sm)
    mask_copy.start()
    for c in range(min(_DEPTH, _NC)):
        in_copy(c).start()
    mask_copy.wait()

    for c in range(_NC):
        slot = c % _NB
        in_copy(c).wait()
        if c >= _NB:
            out_copy(c - _NB).wait()
        mblk = mbuf[:, c * _CW:(c + 1) * _CW]
        ob[slot] = jnp.where(mblk != 0, xb[slot], 0.0)
        out_copy(c).start()
        if c + _DEPTH < _NC:
            in_copy(c + _DEPTH).start()

    for c in range(max(_NC - _NB, 0), _NC):
        out_copy(c).wait()


def kernel(x, mask):
    mask8 = mask.view(jnp.int8)
    return pl.pallas_call(
        _body,
        in_specs=[
            pl.BlockSpec(memory_space=pltpu.MemorySpace.HBM),
            pl.BlockSpec(memory_space=pltpu.MemorySpace.HBM),
        ],
        out_specs=pl.BlockSpec(memory_space=pltpu.MemorySpace.HBM),
        out_shape=jax.ShapeDtypeStruct((_B, _N), jnp.float32),
        scratch_shapes=[
            pltpu.VMEM((_NB, _B, _CW), jnp.float32),
            pltpu.VMEM((_B, _N), jnp.int8),
            pltpu.VMEM((_NB, _B, _CW), jnp.float32),
            pltpu.SemaphoreType.DMA((_NB,)),
            pltpu.SemaphoreType.DMA(()),
            pltpu.SemaphoreType.DMA((_NB,)),
        ],
    )(x, mask8)


# speedup vs baseline: 1.0249x; 1.0249x over previous
import jax
import jax.numpy as jnp
from jax.experimental import pallas as pl
from jax.experimental.pallas import tpu as pltpu

_B = 128
_N = 32768
_CW = 16384
_NC = _N // _CW
_NB = 2
_DEPTH = 2


def _body(x_hbm, m8_hbm, o_hbm, xb, mbuf, ob, sx, sm, so):

    def in_copy(c):
        slot = c % _NB
        return pltpu.make_async_copy(
            x_hbm.at[:, pl.ds(c * _CW, _CW)], xb.at[slot], sx.at[slot])

    def out_copy(c):
        slot = c % _NB
        return pltpu.make_async_copy(
            ob.at[slot], o_hbm.at[:, pl.ds(c * _CW, _CW)], so.at[slot])

    mask_copy = pltpu.make_async_copy(m8_hbm, mbuf, sm)
    mask_copy.start()
    for c in range(min(_DEPTH, _NC)):
        in_copy(c).start()
    mask_copy.wait()

    for c in range(_NC):
        slot = c % _NB
        in_copy(c).wait()
        if c >= _NB:
            out_copy(c - _NB).wait()
        mblk = mbuf[:, c * _CW:(c + 1) * _CW]
        ob[slot] = jnp.where(mblk != 0, xb[slot], 0.0)
        out_copy(c).start()
        if c + _DEPTH < _NC:
            in_copy(c + _DEPTH).start()

    for c in range(max(_NC - _NB, 0), _NC):
        out_copy(c).wait()


def kernel(x, mask):
    mask8 = mask.view(jnp.int8)
    return pl.pallas_call(
        _body,
        in_specs=[
            pl.BlockSpec(memory_space=pltpu.MemorySpace.HBM),
            pl.BlockSpec(memory_space=pltpu.MemorySpace.HBM),
        ],
        out_specs=pl.BlockSpec(memory_space=pltpu.MemorySpace.HBM),
        out_shape=jax.ShapeDtypeStruct((_B, _N), jnp.float32),
        scratch_shapes=[
            pltpu.VMEM((_NB, _B, _CW), jnp.float32),
            pltpu.VMEM((_B, _N), jnp.int8),
            pltpu.VMEM((_NB, _B, _CW), jnp.float32),
            pltpu.SemaphoreType.DMA((_NB,)),
            pltpu.SemaphoreType.DMA(()),
            pltpu.SemaphoreType.DMA((_NB,)),
        ],
    )(x, mask8)
